# Initial kernel scaffold; baseline (speedup 1.0000x reference)
import jax
import jax.numpy as jnp
from jax.experimental import pallas as pl


def kernel(prop, type_emb, count_val, count_bit, fp_pair, fp_bit, fp_val):
    raise NotImplementedError("write your pallas kernel here")


# speedup vs baseline: 3.4159x; 3.4159x over previous
import functools

import jax
import jax.numpy as jnp
from jax import lax
from jax.experimental import pallas as pl
from jax.experimental.pallas import tpu as pltpu
from jax.experimental.pallas import tpu_sc as plsc

B = 1024
COUNT_DIM = 8
NUM_PROPS = 1032
FP_DIM = NUM_PROPS - COUNT_DIM
N_EMBD = 64
K = NUM_PROPS // 2
KP = 528
NC, NS = 2, 16
NW = NC * NS
BPW = B // NW

GATHER_CHUNKS = ((0, 120), (120, 120), (240, 120), (360, 120), (480, 48))

_mesh = plsc.VectorSubcoreMesh(core_axis_name="c", subcore_axis_name="s")


@functools.partial(
    pl.kernel,
    mesh=_mesh,
    out_type=jax.ShapeDtypeStruct((B * K, 2 * N_EMBD), jnp.float32),
    scratch_types=[
        pltpu.VMEM((KP,), jnp.int32),
        pltpu.VMEM((KP,), jnp.int32),
        pltpu.VMEM((KP, 2 * N_EMBD), jnp.float32),
        pltpu.SemaphoreType.DMA,
    ],
)
def _sc_embed(q_hbm, table_hbm, out_hbm, q_v, idx_v, row_v, sem):
    wid = lax.axis_index("c") * NS + lax.axis_index("s")
    row0 = wid * BPW

    def per_row(i, carry):
        b = row0 + i
        pltpu.sync_copy(q_hbm.at[pl.ds(b * KP, KP)], q_v)

        def per_chunk(c, carry2):
            k = c * 16 + lax.broadcasted_iota(jnp.int32, (16,), 0)
            ks = jnp.where(k < K, k, K - 1)
            idx_v[pl.ds(c * 16, 16)] = q_v[pl.ds(c * 16, 16)] * K + ks
            return carry2

        lax.fori_loop(0, KP // 16, per_chunk, 0)

        gathers = [
            pltpu.async_copy(
                table_hbm.at[idx_v.at[pl.ds(off, n)]],
                row_v.at[pl.ds(off, n)],
                sem,
            )
            for off, n in GATHER_CHUNKS
        ]
        for cp in gathers:
            cp.wait()

        scatters = []
        for c in range(KP // 16):
            k = c * 16 + lax.broadcasted_iota(jnp.int32, (16,), 0)
            oidx = b * K + jnp.where(k < K, k, K - 1)
            scatters.append(
                pltpu.async_copy(row_v.at[pl.ds(c * 16, 16)],
                                 out_hbm.at[oidx], sem))
        for cp in scatters:
            cp.wait()
        return carry

    lax.fori_loop(0, BPW, per_row, 0)


def _build_table(type_emb, count_val, count_bit, fp_pair, fp_bit, fp_val):
    base_c = count_val[0] + count_bit + type_emb[0]
    base_f = (fp_val[0]
              + jnp.repeat(fp_pair, 2, axis=0)
              + jnp.tile(fp_bit, (FP_DIM // 2, 1))
              + type_emb[1])
    base = jnp.concatenate([base_c, base_f], axis=0)
    delta_c = jnp.broadcast_to(count_val[1] - count_val[0],
                               (COUNT_DIM, N_EMBD))
    delta_f = jnp.broadcast_to(fp_val[1] - fp_val[0], (FP_DIM, N_EMBD))
    delta = jnp.concatenate([delta_c, delta_f], axis=0)
    full = jnp.stack([base, base + delta])
    even = full[:, 0::2, :]
    odd = full[:, 1::2, :]
    paired = jnp.concatenate([
        jnp.broadcast_to(even[:, None], (2, 2, K, N_EMBD)),
        jnp.broadcast_to(odd[None, :], (2, 2, K, N_EMBD)),
    ], axis=-1)
    return paired.reshape(4 * K, 2 * N_EMBD)


def kernel(prop, type_emb, count_val, count_bit, fp_pair, fp_bit, fp_val):
    table = _build_table(type_emb, count_val, count_bit, fp_pair, fp_bit,
                         fp_val)
    q = 2 * prop[:, 0::2] + prop[:, 1::2]
    q = jnp.concatenate(
        [q, jnp.broadcast_to(q[:, K - 1:K], (B, KP - K))], axis=1)
    out = _sc_embed(q.reshape(-1), table)
    return out.reshape(B, NUM_PROPS, N_EMBD)


# speedup vs baseline: 8.4066x; 2.4611x over previous
import functools

import jax
import jax.numpy as jnp
from jax import lax
from jax.experimental import pallas as pl
from jax.experimental.pallas import tpu as pltpu
from jax.experimental.pallas import tpu_sc as plsc

B = 1024
COUNT_DIM = 8
NUM_PROPS = 1032
FP_DIM = NUM_PROPS - COUNT_DIM
N_EMBD = 64
K = NUM_PROPS // 2
KP = 528
NC, NS = 2, 16
NW = NC * NS
BPW = B // NW

GATHER_CHUNKS = ((0, 120), (120, 120), (240, 120), (360, 120), (480, 48))

_mesh = plsc.VectorSubcoreMesh(core_axis_name="c", subcore_axis_name="s")


@functools.partial(
    pl.kernel,
    mesh=_mesh,
    out_type=jax.ShapeDtypeStruct((B, K, 2 * N_EMBD), jnp.float32),
    scratch_types=[
        pltpu.VMEM((KP,), jnp.int32),
        pltpu.VMEM((KP,), jnp.int32),
        pltpu.VMEM((KP, 2 * N_EMBD), jnp.float32),
        pltpu.VMEM_SHARED((4 * K, 2 * N_EMBD), jnp.float32),
        pltpu.SemaphoreType.DMA,
    ],
)
def _sc_embed(q_hbm, table_hbm, out_hbm, q_v, idx_v, row_v, table_s, sem):
    sid = lax.axis_index("s")
    wid = lax.axis_index("c") * NS + sid
    row0 = wid * BPW

    @pl.when(sid == 0)
    def _():
        pltpu.sync_copy(table_hbm, table_s)

    plsc.subcore_barrier()

    def per_row(i, carry):
        b = row0 + i
        pltpu.sync_copy(q_hbm.at[pl.ds(b * KP, KP)], q_v)

        def per_chunk(c, carry2):
            k = c * 16 + lax.broadcasted_iota(jnp.int32, (16,), 0)
            ks = jnp.where(k < K, k, K - 1)
            idx_v[pl.ds(c * 16, 16)] = q_v[pl.ds(c * 16, 16)] * K + ks
            return carry2

        lax.fori_loop(0, KP // 16, per_chunk, 0)

        gathers = [
            pltpu.async_copy(
                table_s.at[idx_v.at[pl.ds(off, n)]],
                row_v.at[pl.ds(off, n)],
                sem,
            )
            for off, n in GATHER_CHUNKS
        ]
        for cp in gathers:
            cp.wait()

        pltpu.sync_copy(row_v.at[pl.ds(0, K)], out_hbm.at[b])
        return carry

    lax.fori_loop(0, BPW, per_row, 0)


def _build_table(type_emb, count_val, count_bit, fp_pair, fp_bit, fp_val):
    base_c = count_val[0] + count_bit + type_emb[0]
    base_f = (fp_val[0]
              + jnp.repeat(fp_pair, 2, axis=0)
              + jnp.tile(fp_bit, (FP_DIM // 2, 1))
              + type_emb[1])
    base = jnp.concatenate([base_c, base_f], axis=0)
    delta_c = jnp.broadcast_to(count_val[1] - count_val[0],
                               (COUNT_DIM, N_EMBD))
    delta_f = jnp.broadcast_to(fp_val[1] - fp_val[0], (FP_DIM, N_EMBD))
    delta = jnp.concatenate([delta_c, delta_f], axis=0)
    full = jnp.stack([base, base + delta])
    even = full[:, 0::2, :]
    odd = full[:, 1::2, :]
    paired = jnp.concatenate([
        jnp.broadcast_to(even[:, None], (2, 2, K, N_EMBD)),
        jnp.broadcast_to(odd[None, :], (2, 2, K, N_EMBD)),
    ], axis=-1)
    return paired.reshape(4 * K, 2 * N_EMBD)


def kernel(prop, type_emb, count_val, count_bit, fp_pair, fp_bit, fp_val):
    table = _build_table(type_emb, count_val, count_bit, fp_pair, fp_bit,
                         fp_val)
    q = 2 * prop[:, 0::2] + prop[:, 1::2]
    q = jnp.concatenate(
        [q, jnp.broadcast_to(q[:, K - 1:K], (B, KP - K))], axis=1)
    out = _sc_embed(q.reshape(-1), table)
    return out.reshape(B, NUM_PROPS, N_EMBD)
